# baseline (speedup 1.0000x reference)
import functools

import jax
import jax.numpy as jnp
from jax import lax
from jax.experimental import pallas as pl
from jax.experimental.pallas import tpu as pltpu
from jax.experimental.pallas import tpu_sc as plsc


@functools.lru_cache(maxsize=None)
def _make_embed_kernel(V, D, B, S):
    info = plsc.get_sparse_core_info()
    NC, NS, L = info.num_cores, info.num_subcores, info.num_lanes
    NW = NC * NS
    SGRP = 4
    BCH = NW // SGRP
    NBH = B // 128 // BCH
    Bc = NBH * 128
    T = S // SGRP
    DH = D // 8
    NB = B // 128
    assert D % L == 0 and S % 8 == 0 and T % 2 == 0 and B % (128 * BCH) == 0

    mesh = plsc.VectorSubcoreMesh(core_axis_name="c", subcore_axis_name="s")

    @functools.partial(
        pl.kernel,
        mesh=mesh,
        compiler_params=pltpu.CompilerParams(
            use_tc_tiling_on_sc=False, needs_layout_passes=False
        ),
        out_type=jax.ShapeDtypeStruct((B * S * D // 128, 128), jnp.float32),
        scratch_types=[
            pltpu.VMEM((2, NBH, 128), jnp.int32),
            pltpu.VMEM((2, Bc, D), jnp.float32),
            pltpu.VMEM((DH * 40, 129), jnp.float32),
            pltpu.VMEM((S, D), jnp.float32),
            pltpu.SemaphoreType.DMA,
            pltpu.SemaphoreType.DMA,
            pltpu.SemaphoreType.DMA,
            pltpu.SemaphoreType.DMA,
            pltpu.SemaphoreType.DMA,
        ],
    )
    def embed(table_hbm, x2_hbm, pos_hbm, out_hbm, idx_v, rows_v, tv, pos_v,
              isem0, isem1, gsem0, gsem1, osem):
        iota = lax.iota(jnp.int32, L)
        dh_base = iota // 8
        dl_base = iota % 8
        wid = lax.axis_index("s") * NC + lax.axis_index("c")
        sgrp = wid // BCH
        bh0 = (wid % BCH) * NBH
        isems = (isem0, isem1)
        gsems = (gsem0, gsem1)

        def stage_idx(t, buf, sem):
            s = sgrp * T + t
            xrow0 = ((s // 8) * NB + bh0) * 8 + (s % 8)
            for j in range(NBH):
                pltpu.async_copy(x2_hbm.at[xrow0 + j * 8], idx_v.at[buf, j], sem)

        def drain_idx(buf, sem):
            for j in range(NBH):
                pltpu.make_async_copy(x2_hbm.at[0], idx_v.at[buf, j], sem).wait()

        def fire_gathers(buf, sem):
            for j in range(NBH):
                pltpu.async_copy(
                    table_hbm.at[idx_v.at[buf, j]],
                    rows_v.at[buf, pl.ds(j * 128, 128)],
                    sem,
                )

        def drain_gathers(buf, sem):
            pltpu.make_async_copy(
                table_hbm.at[pl.ds(0, Bc)], rows_v.at[buf], sem
            ).wait()

        def transpose(t, buf):
            s = sgrp * T + t
            pos_u = [pos_v[s, pl.ds(u * L, L)] for u in range(D // L)]

            @plsc.parallel_loop(0, Bc, 1, unroll=8)
            def b_body(b):
                bh_loc = b // 128
                bl16 = jnp.full((L,), b % 128, jnp.int32)
                row16 = bh_loc * 8 + dl_base
                for u in range(D // L):
                    val = rows_v[buf, b, pl.ds(u * L, L)] + pos_u[u]
                    plsc.store_scatter(
                        tv, [(2 * u + dh_base) * 40 + row16, bl16], val)

        def fire_out(t):
            s = sgrp * T + t
            for dh in range(DH):
                pltpu.async_copy(
                    tv.at[pl.ds(dh * 40, NBH * 8), pl.ds(0, 128)],
                    out_hbm.at[pl.ds(((s * DH + dh) * NB + bh0) * 8, NBH * 8)],
                    osem,
                )

        def drain_out():
            for dh in range(DH):
                pltpu.make_async_copy(
                    out_hbm.at[pl.ds(0, NBH * 8)],
                    tv.at[pl.ds(dh * 40, NBH * 8), pl.ds(0, 128)], osem
                ).wait()

        pltpu.sync_copy(pos_hbm, pos_v)
        stage_idx(0, 0, isem0)
        drain_idx(0, isem0)
        fire_gathers(0, gsem0)
        stage_idx(1, 1, isem1)

        def pair_body(m, carry):
            t0 = 2 * m

            def half(t, buf):
                nbuf = 1 - buf
                drain_idx(nbuf, isems[nbuf])
                fire_gathers(nbuf, gsems[nbuf])
                drain_gathers(buf, gsems[buf])
                stage_idx(lax.rem(t + 2, T), buf, isems[buf])

                @pl.when(t > 0)
                def _():
                    drain_out()

                transpose(t, buf)
                fire_out(t)

            half(t0, 0)
            half(t0 + 1, 1)
            return carry

        lax.fori_loop(0, T // 2, pair_body, 0)
        drain_out()
        drain_idx(1, isem1)
        drain_gathers(0, gsem0)

    return embed


def kernel(x, token_table, pos_table):
    B, S = x.shape
    V, D = token_table.shape
    x2 = (
        x.astype(jnp.int32)
        .reshape(B // 128, 128, S // 8, 8)
        .transpose(2, 0, 3, 1)
        .reshape(B * S // 128, 128)
    )
    embed = _make_embed_kernel(V, D, B, S)
    out2 = embed(token_table, x2, pos_table)
    return (
        out2.reshape(S, D // 8, B // 128, 8, 128)
        .transpose(2, 4, 0, 1, 3)
        .reshape(B, S, D)
    )


# speedup vs baseline: 1.0999x; 1.0999x over previous
import functools

import jax
import jax.numpy as jnp
from jax import lax
from jax.experimental import pallas as pl
from jax.experimental.pallas import tpu as pltpu
from jax.experimental.pallas import tpu_sc as plsc


@functools.lru_cache(maxsize=None)
def _make_embed_kernel(V, D, B, S):
    info = plsc.get_sparse_core_info()
    NC, NS, L = info.num_cores, info.num_subcores, info.num_lanes
    NW = NC * NS
    SGRP = 2
    BCH = NW // SGRP
    NBH = B // 128 // BCH
    Bc = NBH * 128
    T = S // SGRP
    DH = D // 8
    NB = B // 128
    SKR = NBH * 8 + 8
    assert D % L == 0 and S % 8 == 0 and T % 2 == 0 and B % (128 * BCH) == 0

    mesh = plsc.VectorSubcoreMesh(core_axis_name="c", subcore_axis_name="s")

    @functools.partial(
        pl.kernel,
        mesh=mesh,
        compiler_params=pltpu.CompilerParams(
            use_tc_tiling_on_sc=False, needs_layout_passes=False
        ),
        out_type=jax.ShapeDtypeStruct((B * S * D // 128, 128), jnp.float32),
        scratch_types=[
            pltpu.VMEM((2, NBH, 128), jnp.int32),
            pltpu.VMEM((2, Bc, 128), jnp.float32),
            pltpu.VMEM((DH * (NBH * 8 + 8), 129), jnp.float32),
            pltpu.VMEM((S, D), jnp.float32),
            pltpu.SemaphoreType.DMA,
            pltpu.SemaphoreType.DMA,
            pltpu.SemaphoreType.DMA,
            pltpu.SemaphoreType.DMA,
            pltpu.SemaphoreType.DMA,
        ],
    )
    def embed(table_hbm, x2_hbm, pos_hbm, out_hbm, idx_v, rows_v, tv, pos_v,
              isem0, isem1, gsem0, gsem1, osem):
        iota = lax.iota(jnp.int32, L)
        dh_base = iota // 8
        dl_base = iota % 8
        wid = lax.axis_index("s") * NC + lax.axis_index("c")
        sgrp = wid // BCH
        bh0 = (wid % BCH) * NBH
        isems = (isem0, isem1)
        gsems = (gsem0, gsem1)

        def stage_idx(t, buf, sem):
            s = sgrp * T + t
            xrow0 = ((s // 8) * NB + bh0) * 8 + (s % 8)
            for j in range(NBH):
                pltpu.async_copy(x2_hbm.at[xrow0 + j * 8], idx_v.at[buf, j], sem)

        def drain_idx(buf, sem):
            for j in range(NBH):
                pltpu.make_async_copy(x2_hbm.at[0], idx_v.at[buf, j], sem).wait()

        def fire_gathers(buf, sem):
            for j in range(NBH):
                pltpu.async_copy(
                    table_hbm.at[idx_v.at[buf, j]],
                    rows_v.at[buf, pl.ds(j * 128, 128)],
                    sem,
                )

        def drain_gathers(buf, sem):
            pltpu.make_async_copy(
                table_hbm.at[pl.ds(0, Bc)], rows_v.at[buf], sem
            ).wait()

        def transpose(t, buf):
            s = sgrp * T + t
            pos_u = [pos_v[s, pl.ds(u * L, L)] for u in range(D // L)]

            @plsc.parallel_loop(0, Bc, 1, unroll=8)
            def b_body(b):
                bh_loc = b // 128
                bl16 = jnp.full((L,), b % 128, jnp.int32)
                row16 = bh_loc * 8 + dl_base
                for u in range(D // L):
                    val = rows_v[buf, b, pl.ds(u * L, L)] + pos_u[u]
                    plsc.store_scatter(
                        tv, [(2 * u + dh_base) * SKR + row16, bl16], val)

        def fire_out(t):
            s = sgrp * T + t
            for dh in range(DH):
                pltpu.async_copy(
                    tv.at[pl.ds(dh * SKR, NBH * 8), pl.ds(0, 128)],
                    out_hbm.at[pl.ds(((s * DH + dh) * NB + bh0) * 8, NBH * 8)],
                    osem,
                )

        def drain_out():
            for dh in range(DH):
                pltpu.make_async_copy(
                    out_hbm.at[pl.ds(0, NBH * 8)],
                    tv.at[pl.ds(dh * SKR, NBH * 8), pl.ds(0, 128)], osem
                ).wait()

        pltpu.sync_copy(pos_hbm, pos_v)
        stage_idx(0, 0, isem0)
        drain_idx(0, isem0)
        fire_gathers(0, gsem0)
        stage_idx(1, 1, isem1)

        def pair_body(m, carry):
            t0 = 2 * m

            def half(t, buf):
                nbuf = 1 - buf
                drain_idx(nbuf, isems[nbuf])
                fire_gathers(nbuf, gsems[nbuf])
                drain_gathers(buf, gsems[buf])
                stage_idx(lax.rem(t + 2, T), buf, isems[buf])

                @pl.when(t > 0)
                def _():
                    drain_out()

                transpose(t, buf)
                fire_out(t)

            half(t0, 0)
            half(t0 + 1, 1)
            return carry

        lax.fori_loop(0, T // 2, pair_body, 0)
        drain_out()
        drain_idx(1, isem1)
        drain_gathers(0, gsem0)

    return embed


@functools.lru_cache(maxsize=None)
def _make_detile(V, D):
    VC = 2048
    G = -(-V // VC)

    def body(tT_ref, out_ref):
        out_ref[:, pl.ds(0, D)] = tT_ref[...].T

    return pl.pallas_call(
        body,
        grid=(G,),
        in_specs=[pl.BlockSpec((D, VC), lambda i: (0, i))],
        out_specs=pl.BlockSpec((VC, 128), lambda i: (i, 0)),
        out_shape=jax.ShapeDtypeStruct((V, 128), jnp.float32),
    )


def kernel(x, token_table, pos_table):
    B, S = x.shape
    V, D = token_table.shape
    x2 = (
        x.astype(jnp.int32)
        .reshape(B // 128, 128, S // 8, 8)
        .transpose(2, 0, 3, 1)
        .reshape(B * S // 128, 128)
    )
    t128 = _make_detile(V, D)(token_table.T)
    embed = _make_embed_kernel(V, D, B, S)
    out2 = embed(t128, x2, pos_table)
    return (
        out2.reshape(S, D // 8, B // 128, 8, 128)
        .transpose(2, 4, 0, 1, 3)
        .reshape(B, S, D)
    )
